# baseline (speedup 1.0000x reference)
import functools

import jax
import jax.numpy as jnp
from jax import lax
from jax.experimental import pallas as pl
from jax.experimental.pallas import tpu as pltpu
from jax.experimental.pallas import tpu_sc as plsc

_N = 10000
_E = 320000
_DF = 128
_H = 64
_OUT = 3
_P = 4

_NP = 10240
_NC = 2
_NS = 16
_NW = _NC * _NS
_RPS = _NP // _NS

_EW = _E // _NW
_CH = 80
_NCHUNK = _EW // _CH

_DEGW = 8
_DLAG = 8
_NBUF = 4

_mesh = plsc.VectorSubcoreMesh(core_axis_name="c", subcore_axis_name="s")
_sc_params = pltpu.CompilerParams(use_tc_tiling_on_sc=False)


def _deg_body(src3, dst3, ones_hbm, zdeg, degp, sdeg, idx_v, ones_v, sem):
    cid = lax.axis_index("c")
    sid = lax.axis_index("s")
    zslab = pl.ds(sid * _RPS, _RPS)
    pltpu.sync_copy(zdeg.at[zslab], sdeg.at[zslab])
    pltpu.sync_copy(ones_hbm, ones_v)

    @pl.when(cid == 0)
    def _():
        pltpu.sync_copy(src3.at[2 * sid], idx_v.at[0])
        pltpu.sync_copy(src3.at[2 * sid + 1], idx_v.at[1])

    @pl.when(cid == 1)
    def _():
        pltpu.sync_copy(dst3.at[2 * sid], idx_v.at[0])
        pltpu.sync_copy(dst3.at[2 * sid + 1], idx_v.at[1])

    plsc.subcore_barrier()
    for s in range(2):
        islab = idx_v.at[s]

        def body(i, carry):
            pltpu.async_copy(ones_v, sdeg.at[islab.at[i]], sem, add=True)

            @pl.when(i >= _DLAG)
            def _():
                pltpu.make_async_copy(ones_v, sdeg.at[islab.at[i - _DLAG]],
                                      sem).wait()

            return carry

        lax.fori_loop(0, _NCHUNK, body, 0)
        for i in range(_NCHUNK - _DLAG, _NCHUNK):
            pltpu.make_async_copy(ones_v, sdeg.at[islab.at[i]], sem).wait()
    plsc.subcore_barrier()
    pltpu.sync_copy(sdeg.at[zslab], degp.at[cid, zslab])


_deg_call = pl.kernel(
    _deg_body,
    out_type=jax.ShapeDtypeStruct((_NC, _NP, _DEGW), jnp.float32),
    mesh=_mesh,
    scratch_types=[
        pltpu.VMEM_SHARED((_NP, _DEGW), jnp.float32),
        pltpu.VMEM((2, _NCHUNK, _CH), jnp.int32),
        pltpu.VMEM((_CH, _DEGW), jnp.float32),
        pltpu.SemaphoreType.DMA,
    ],
    compiler_params=_sc_params,
)


def _agg_body(hs, src3, dst3, zagg, aggp, shs, sagg, src_v, dst_v, *rest):
    rows = rest[:_NBUF]
    sems = rest[_NBUF:]
    cid = lax.axis_index("c")
    sid = lax.axis_index("s")
    wid = sid * _NC + cid
    zslab = pl.ds(sid * _RPS, _RPS)
    pltpu.sync_copy(hs.at[zslab], shs.at[zslab])
    pltpu.sync_copy(zagg.at[zslab], sagg.at[zslab])
    pltpu.sync_copy(src3.at[wid], src_v)
    pltpu.sync_copy(dst3.at[wid], dst_v)
    plsc.subcore_barrier()

    def gather(i, buf, sem):
        pltpu.async_copy(shs.at[src_v.at[i]], buf, sem)

    def wait(i, buf, sem):
        pltpu.make_async_copy(shs.at[src_v.at[i]], buf, sem).wait()

    def scatter(i, buf):
        pltpu.sync_copy(buf, sagg.at[dst_v.at[i]], add=True)

    bufs = list(zip(rows, sems))
    for b in range(_NBUF):
        gather(b, *bufs[b])

    def body(j, carry):
        for b in range(_NBUF):
            i = _NBUF * j + b
            wait(i, *bufs[b])
            scatter(i, bufs[b][0])

            @pl.when(i + _NBUF < _NCHUNK)
            def _():
                gather(i + _NBUF, *bufs[b])

        return carry

    lax.fori_loop(0, _NCHUNK // _NBUF, body, 0)
    for i in range((_NCHUNK // _NBUF) * _NBUF, _NCHUNK):
        wait(i, *bufs[i % _NBUF])
        scatter(i, bufs[i % _NBUF][0])
    plsc.subcore_barrier()
    pltpu.sync_copy(sagg.at[zslab], aggp.at[cid, zslab])


_agg_call = pl.kernel(
    _agg_body,
    out_type=jax.ShapeDtypeStruct((_NC, _NP, _H), jnp.float32),
    mesh=_mesh,
    scratch_types=[
        pltpu.VMEM_SHARED((_NP, _H), jnp.float32),
        pltpu.VMEM_SHARED((_NP, _H), jnp.float32),
        pltpu.VMEM((_NCHUNK, _CH), jnp.int32),
        pltpu.VMEM((_NCHUNK, _CH), jnp.int32),
    ] + [pltpu.VMEM((_CH, _H), jnp.float32)] * _NBUF
      + [pltpu.SemaphoreType.DMA] * _NBUF,
    compiler_params=_sc_params,
)


def _silu(x):
    return x * jax.nn.sigmoid(x)


def _enc_body(x_ref, Wn_ref, bn_ref, gn_ref, betan_ref, degp_ref,
              h_ref, hs_ref, ns_ref, nd_ref):
    t = jnp.dot(x_ref[...], Wn_ref[...], preferred_element_type=jnp.float32)
    t = _silu(t + bn_ref[...])
    mu = jnp.mean(t, axis=-1, keepdims=True)
    var = jnp.mean((t - mu) * (t - mu), axis=-1, keepdims=True)
    h = (t - mu) * lax.rsqrt(var + 1e-5) * gn_ref[...] + betan_ref[...]
    inv = 1.0 / _DEGW
    dego = jnp.sum(degp_ref[0], axis=-1, keepdims=True) * inv
    degi = jnp.sum(degp_ref[1], axis=-1, keepdims=True) * inv
    ns = lax.rsqrt(jnp.maximum(dego, 1.0))
    nd = lax.rsqrt(jnp.maximum(degi, 1.0))
    h_ref[...] = h
    hs_ref[...] = h * ns
    ns_ref[...] = ns
    nd_ref[...] = nd


_enc_call = pl.pallas_call(
    _enc_body,
    out_shape=[
        jax.ShapeDtypeStruct((_NP, _H), jnp.float32),
        jax.ShapeDtypeStruct((_NP, _H), jnp.float32),
        jax.ShapeDtypeStruct((_NP, 1), jnp.float32),
        jax.ShapeDtypeStruct((_NP, 1), jnp.float32),
    ],
)


def _layer_body(aggp_ref, nd_ref, ns_ref, h_ref, W_ref, b_ref,
                ho_ref, hso_ref):
    agg = aggp_ref[0] + aggp_ref[1]
    a = agg * nd_ref[...]
    t = jnp.dot(a, W_ref[...], preferred_element_type=jnp.float32)
    h = h_ref[...] + _silu(t + b_ref[...])
    ho_ref[...] = h
    hso_ref[...] = h * ns_ref[...]


_layer_call = pl.pallas_call(
    _layer_body,
    out_shape=[
        jax.ShapeDtypeStruct((_NP, _H), jnp.float32),
        jax.ShapeDtypeStruct((_NP, _H), jnp.float32),
    ],
)


def _final_body(aggp_ref, nd_ref, h_ref, W_ref, b_ref,
                Wd1_ref, bd1_ref, Wd2_ref, bd2_ref, out_ref):
    agg = aggp_ref[0] + aggp_ref[1]
    a = agg * nd_ref[...]
    t = jnp.dot(a, W_ref[...], preferred_element_type=jnp.float32)
    h = h_ref[...] + _silu(t + b_ref[...])
    d = _silu(jnp.dot(h, Wd1_ref[...], preferred_element_type=jnp.float32)
              + bd1_ref[...])
    out_ref[...] = (jnp.dot(d, Wd2_ref[...], preferred_element_type=jnp.float32)
                    + bd2_ref[...])


_final_call = pl.pallas_call(
    _final_body,
    out_shape=jax.ShapeDtypeStruct((_NP, _OUT), jnp.float32),
)


def kernel(node_feats, edge_feats, edge_index, Wn, bn, gn, betan, We, be, ge,
           betae, convW, convb, Wd1, bd1, Wd2, bd2):
    src3 = edge_index[0].reshape(_NW, _NCHUNK, _CH)
    dst3 = edge_index[1].reshape(_NW, _NCHUNK, _CH)
    x_p = jnp.pad(node_feats, ((0, _NP - _N), (0, 0)))
    zagg = jnp.zeros((_NP, _H), jnp.float32)
    zdeg = jnp.zeros((_NP, _DEGW), jnp.float32)
    ones_hbm = jnp.ones((_CH, _DEGW), jnp.float32)

    degp = _deg_call(src3, dst3, ones_hbm, zdeg)
    h, hs, ns, nd = _enc_call(x_p, Wn, bn, gn, betan, degp)
    for i in range(_P - 1):
        aggp = _agg_call(hs, src3, dst3, zagg)
        h, hs = _layer_call(aggp, nd, ns, h, convW[i], convb[i])
    aggp = _agg_call(hs, src3, dst3, zagg)
    out_p = _final_call(aggp, nd, h, convW[_P - 1], convb[_P - 1],
                        Wd1, bd1, Wd2, bd2)
    return out_p[:_N]


# speedup vs baseline: 1.2305x; 1.2305x over previous
import functools

import jax
import jax.numpy as jnp
from jax import lax
from jax.experimental import pallas as pl
from jax.experimental.pallas import tpu as pltpu
from jax.experimental.pallas import tpu_sc as plsc

_N = 10000
_E = 320000
_DF = 128
_H = 64
_OUT = 3
_P = 4

_NP = 10240
_NC = 2
_NS = 16
_NW = _NC * _NS
_RPS = _NP // _NS

_EW = _E // _NW
_CH = 80
_NCHUNK = _EW // _CH

_DEGW = 8
_DLAG = 8
_NBUF = 4

_mesh = plsc.VectorSubcoreMesh(core_axis_name="c", subcore_axis_name="s")
_sc_params = pltpu.CompilerParams(use_tc_tiling_on_sc=False)


def _deg_body(src3, dst3, ones_hbm, zdeg, degp, sdeg, idx_v, ones_v, sem):
    cid = lax.axis_index("c")
    sid = lax.axis_index("s")
    zslab = pl.ds(sid * _RPS, _RPS)
    pltpu.sync_copy(zdeg.at[zslab], sdeg.at[zslab])
    pltpu.sync_copy(ones_hbm, ones_v)

    @pl.when(cid == 0)
    def _():
        pltpu.sync_copy(src3.at[2 * sid], idx_v.at[0])
        pltpu.sync_copy(src3.at[2 * sid + 1], idx_v.at[1])

    @pl.when(cid == 1)
    def _():
        pltpu.sync_copy(dst3.at[2 * sid], idx_v.at[0])
        pltpu.sync_copy(dst3.at[2 * sid + 1], idx_v.at[1])

    plsc.subcore_barrier()
    for s in range(2):
        islab = idx_v.at[s]

        def body(i, carry):
            pltpu.async_copy(ones_v, sdeg.at[islab.at[i]], sem, add=True)

            @pl.when(i >= _DLAG)
            def _():
                pltpu.make_async_copy(ones_v, sdeg.at[islab.at[i - _DLAG]],
                                      sem).wait()

            return carry

        lax.fori_loop(0, _NCHUNK, body, 0)
        for i in range(_NCHUNK - _DLAG, _NCHUNK):
            pltpu.make_async_copy(ones_v, sdeg.at[islab.at[i]], sem).wait()
    plsc.subcore_barrier()
    pltpu.sync_copy(sdeg.at[zslab], degp.at[cid, zslab])


_deg_call = pl.kernel(
    _deg_body,
    out_type=jax.ShapeDtypeStruct((_NC, _NP, _DEGW), jnp.float32),
    mesh=_mesh,
    scratch_types=[
        pltpu.VMEM_SHARED((_NP, _DEGW), jnp.float32),
        pltpu.VMEM((2, _NCHUNK, _CH), jnp.int32),
        pltpu.VMEM((_CH, _DEGW), jnp.float32),
        pltpu.SemaphoreType.DMA,
    ],
    compiler_params=_sc_params,
)


def _agg_body(hs, src3, dst3, zagg, aggp, sagg, src_v, dst_v, *rest):
    rows = rest[:_NBUF]
    sems = rest[_NBUF:]
    cid = lax.axis_index("c")
    sid = lax.axis_index("s")
    wid = sid * _NC + cid
    zslab = pl.ds(sid * _RPS, _RPS)
    pltpu.sync_copy(zagg.at[zslab], sagg.at[zslab])
    pltpu.sync_copy(src3.at[wid], src_v)
    pltpu.sync_copy(dst3.at[wid], dst_v)
    plsc.subcore_barrier()

    def gather(i, buf, sem):
        pltpu.async_copy(hs.at[src_v.at[i]], buf, sem)

    def wait(i, buf, sem):
        pltpu.make_async_copy(hs.at[src_v.at[i]], buf, sem).wait()

    def scatter(i, buf):
        pltpu.sync_copy(buf, sagg.at[dst_v.at[i]], add=True)

    bufs = list(zip(rows, sems))
    for b in range(_NBUF):
        gather(b, *bufs[b])

    def body(j, carry):
        for b in range(_NBUF):
            i = _NBUF * j + b
            wait(i, *bufs[b])
            scatter(i, bufs[b][0])

            @pl.when(i + _NBUF < _NCHUNK)
            def _():
                gather(i + _NBUF, *bufs[b])

        return carry

    lax.fori_loop(0, _NCHUNK // _NBUF, body, 0)
    for i in range((_NCHUNK // _NBUF) * _NBUF, _NCHUNK):
        wait(i, *bufs[i % _NBUF])
        scatter(i, bufs[i % _NBUF][0])
    plsc.subcore_barrier()
    pltpu.sync_copy(sagg.at[zslab], aggp.at[cid, zslab])


_agg_call = pl.kernel(
    _agg_body,
    out_type=jax.ShapeDtypeStruct((_NC, _NP, _H), jnp.float32),
    mesh=_mesh,
    scratch_types=[
        pltpu.VMEM_SHARED((_NP, _H), jnp.float32),
        pltpu.VMEM((_NCHUNK, _CH), jnp.int32),
        pltpu.VMEM((_NCHUNK, _CH), jnp.int32),
    ] + [pltpu.VMEM((_CH, _H), jnp.float32)] * _NBUF
      + [pltpu.SemaphoreType.DMA] * _NBUF,
    compiler_params=_sc_params,
)


def _silu(x):
    return x * jax.nn.sigmoid(x)


def _enc_body(x_ref, Wn_ref, bn_ref, gn_ref, betan_ref, degp_ref,
              h_ref, hs_ref, ns_ref, nd_ref):
    t = jnp.dot(x_ref[...], Wn_ref[...], preferred_element_type=jnp.float32)
    t = _silu(t + bn_ref[...])
    mu = jnp.mean(t, axis=-1, keepdims=True)
    var = jnp.mean((t - mu) * (t - mu), axis=-1, keepdims=True)
    h = (t - mu) * lax.rsqrt(var + 1e-5) * gn_ref[...] + betan_ref[...]
    inv = 1.0 / _DEGW
    dego = jnp.sum(degp_ref[0], axis=-1, keepdims=True) * inv
    degi = jnp.sum(degp_ref[1], axis=-1, keepdims=True) * inv
    ns = lax.rsqrt(jnp.maximum(dego, 1.0))
    nd = lax.rsqrt(jnp.maximum(degi, 1.0))
    h_ref[...] = h
    hs_ref[...] = h * ns
    ns_ref[...] = ns
    nd_ref[...] = nd


_enc_call = pl.pallas_call(
    _enc_body,
    out_shape=[
        jax.ShapeDtypeStruct((_NP, _H), jnp.float32),
        jax.ShapeDtypeStruct((_NP, _H), jnp.float32),
        jax.ShapeDtypeStruct((_NP, 1), jnp.float32),
        jax.ShapeDtypeStruct((_NP, 1), jnp.float32),
    ],
)


def _layer_body(aggp_ref, nd_ref, ns_ref, h_ref, W_ref, b_ref,
                ho_ref, hso_ref):
    agg = aggp_ref[0] + aggp_ref[1]
    a = agg * nd_ref[...]
    t = jnp.dot(a, W_ref[...], preferred_element_type=jnp.float32)
    h = h_ref[...] + _silu(t + b_ref[...])
    ho_ref[...] = h
    hso_ref[...] = h * ns_ref[...]


_layer_call = pl.pallas_call(
    _layer_body,
    out_shape=[
        jax.ShapeDtypeStruct((_NP, _H), jnp.float32),
        jax.ShapeDtypeStruct((_NP, _H), jnp.float32),
    ],
)


def _final_body(aggp_ref, nd_ref, h_ref, W_ref, b_ref,
                Wd1_ref, bd1_ref, Wd2_ref, bd2_ref, out_ref):
    agg = aggp_ref[0] + aggp_ref[1]
    a = agg * nd_ref[...]
    t = jnp.dot(a, W_ref[...], preferred_element_type=jnp.float32)
    h = h_ref[...] + _silu(t + b_ref[...])
    d = _silu(jnp.dot(h, Wd1_ref[...], preferred_element_type=jnp.float32)
              + bd1_ref[...])
    out_ref[...] = (jnp.dot(d, Wd2_ref[...], preferred_element_type=jnp.float32)
                    + bd2_ref[...])


_final_call = pl.pallas_call(
    _final_body,
    out_shape=jax.ShapeDtypeStruct((_NP, _OUT), jnp.float32),
)


def kernel(node_feats, edge_feats, edge_index, Wn, bn, gn, betan, We, be, ge,
           betae, convW, convb, Wd1, bd1, Wd2, bd2):
    src3 = edge_index[0].reshape(_NW, _NCHUNK, _CH)
    dst3 = edge_index[1].reshape(_NW, _NCHUNK, _CH)
    x_p = jnp.pad(node_feats, ((0, _NP - _N), (0, 0)))
    zagg = jnp.zeros((_NP, _H), jnp.float32)
    zdeg = jnp.zeros((_NP, _DEGW), jnp.float32)
    ones_hbm = jnp.ones((_CH, _DEGW), jnp.float32)

    degp = _deg_call(src3, dst3, ones_hbm, zdeg)
    h, hs, ns, nd = _enc_call(x_p, Wn, bn, gn, betan, degp)
    for i in range(_P - 1):
        aggp = _agg_call(hs, src3, dst3, zagg)
        h, hs = _layer_call(aggp, nd, ns, h, convW[i], convb[i])
    aggp = _agg_call(hs, src3, dst3, zagg)
    out_p = _final_call(aggp, nd, h, convW[_P - 1], convb[_P - 1],
                        Wd1, bd1, Wd2, bd2)
    return out_p[:_N]


# speedup vs baseline: 1.2508x; 1.0165x over previous
import functools

import jax
import jax.numpy as jnp
from jax import lax
from jax.experimental import pallas as pl
from jax.experimental.pallas import tpu as pltpu
from jax.experimental.pallas import tpu_sc as plsc

_N = 10000
_E = 320000
_DF = 128
_H = 64
_OUT = 3
_P = 4

_NP = 10240
_NC = 2
_NS = 16
_NW = _NC * _NS
_RPS = _NP // _NS

_EW = _E // _NW
_CH = 80
_NCHUNK = _EW // _CH

_DEGW = 8
_DLAG = 8
_NBUF = 6

_mesh = plsc.VectorSubcoreMesh(core_axis_name="c", subcore_axis_name="s")
_sc_params = pltpu.CompilerParams(use_tc_tiling_on_sc=False)


def _deg_body(src3, dst3, ones_hbm, zdeg, degp, sdeg, idx_v, ones_v, sem):
    cid = lax.axis_index("c")
    sid = lax.axis_index("s")
    zslab = pl.ds(sid * _RPS, _RPS)
    pltpu.sync_copy(zdeg.at[zslab], sdeg.at[zslab])
    pltpu.sync_copy(ones_hbm, ones_v)

    @pl.when(cid == 0)
    def _():
        pltpu.sync_copy(src3.at[2 * sid], idx_v.at[0])
        pltpu.sync_copy(src3.at[2 * sid + 1], idx_v.at[1])

    @pl.when(cid == 1)
    def _():
        pltpu.sync_copy(dst3.at[2 * sid], idx_v.at[0])
        pltpu.sync_copy(dst3.at[2 * sid + 1], idx_v.at[1])

    plsc.subcore_barrier()
    for s in range(2):
        islab = idx_v.at[s]

        def body(i, carry):
            pltpu.async_copy(ones_v, sdeg.at[islab.at[i]], sem, add=True)

            @pl.when(i >= _DLAG)
            def _():
                pltpu.make_async_copy(ones_v, sdeg.at[islab.at[i - _DLAG]],
                                      sem).wait()

            return carry

        lax.fori_loop(0, _NCHUNK, body, 0)
        for i in range(_NCHUNK - _DLAG, _NCHUNK):
            pltpu.make_async_copy(ones_v, sdeg.at[islab.at[i]], sem).wait()
    plsc.subcore_barrier()
    pltpu.sync_copy(sdeg.at[zslab], degp.at[cid, zslab])


_deg_call = pl.kernel(
    _deg_body,
    out_type=jax.ShapeDtypeStruct((_NC, _NP, _DEGW), jnp.float32),
    mesh=_mesh,
    scratch_types=[
        pltpu.VMEM_SHARED((_NP, _DEGW), jnp.float32),
        pltpu.VMEM((2, _NCHUNK, _CH), jnp.int32),
        pltpu.VMEM((_CH, _DEGW), jnp.float32),
        pltpu.SemaphoreType.DMA,
    ],
    compiler_params=_sc_params,
)


def _agg_body(hs, src3, dst3, zagg, aggp, sagg, src_v, dst_v, *rest):
    rows = rest[:_NBUF]
    sems = rest[_NBUF:]
    cid = lax.axis_index("c")
    sid = lax.axis_index("s")
    wid = sid * _NC + cid
    zslab = pl.ds(sid * _RPS, _RPS)
    pltpu.sync_copy(zagg.at[zslab], sagg.at[zslab])
    pltpu.sync_copy(src3.at[wid], src_v)
    pltpu.sync_copy(dst3.at[wid], dst_v)
    plsc.subcore_barrier()

    def gather(i, buf, sem):
        pltpu.async_copy(hs.at[src_v.at[i]], buf, sem)

    def wait(i, buf, sem):
        pltpu.make_async_copy(hs.at[src_v.at[i]], buf, sem).wait()

    def scatter(i, buf):
        pltpu.sync_copy(buf, sagg.at[dst_v.at[i]], add=True)

    bufs = list(zip(rows, sems))
    for b in range(_NBUF):
        gather(b, *bufs[b])

    def body(j, carry):
        for b in range(_NBUF):
            i = _NBUF * j + b
            wait(i, *bufs[b])
            scatter(i, bufs[b][0])

            @pl.when(i + _NBUF < _NCHUNK)
            def _():
                gather(i + _NBUF, *bufs[b])

        return carry

    lax.fori_loop(0, _NCHUNK // _NBUF, body, 0)
    for i in range((_NCHUNK // _NBUF) * _NBUF, _NCHUNK):
        wait(i, *bufs[i % _NBUF])
        scatter(i, bufs[i % _NBUF][0])
    plsc.subcore_barrier()
    pltpu.sync_copy(sagg.at[zslab], aggp.at[cid, zslab])


_agg_call = pl.kernel(
    _agg_body,
    out_type=jax.ShapeDtypeStruct((_NC, _NP, _H), jnp.float32),
    mesh=_mesh,
    scratch_types=[
        pltpu.VMEM_SHARED((_NP, _H), jnp.float32),
        pltpu.VMEM((_NCHUNK, _CH), jnp.int32),
        pltpu.VMEM((_NCHUNK, _CH), jnp.int32),
    ] + [pltpu.VMEM((_CH, _H), jnp.float32)] * _NBUF
      + [pltpu.SemaphoreType.DMA] * _NBUF,
    compiler_params=_sc_params,
)


def _silu(x):
    return x * jax.nn.sigmoid(x)


def _enc_body(x_ref, Wn_ref, bn_ref, gn_ref, betan_ref, degp_ref,
              h_ref, hs_ref, ns_ref, nd_ref):
    t = jnp.dot(x_ref[...], Wn_ref[...], preferred_element_type=jnp.float32)
    t = _silu(t + bn_ref[...])
    mu = jnp.mean(t, axis=-1, keepdims=True)
    var = jnp.mean((t - mu) * (t - mu), axis=-1, keepdims=True)
    h = (t - mu) * lax.rsqrt(var + 1e-5) * gn_ref[...] + betan_ref[...]
    inv = 1.0 / _DEGW
    dego = jnp.sum(degp_ref[0], axis=-1, keepdims=True) * inv
    degi = jnp.sum(degp_ref[1], axis=-1, keepdims=True) * inv
    ns = lax.rsqrt(jnp.maximum(dego, 1.0))
    nd = lax.rsqrt(jnp.maximum(degi, 1.0))
    h_ref[...] = h
    hs_ref[...] = h * ns
    ns_ref[...] = ns
    nd_ref[...] = nd


_enc_call = pl.pallas_call(
    _enc_body,
    out_shape=[
        jax.ShapeDtypeStruct((_NP, _H), jnp.float32),
        jax.ShapeDtypeStruct((_NP, _H), jnp.float32),
        jax.ShapeDtypeStruct((_NP, 1), jnp.float32),
        jax.ShapeDtypeStruct((_NP, 1), jnp.float32),
    ],
)


def _layer_body(aggp_ref, nd_ref, ns_ref, h_ref, W_ref, b_ref,
                ho_ref, hso_ref):
    agg = aggp_ref[0] + aggp_ref[1]
    a = agg * nd_ref[...]
    t = jnp.dot(a, W_ref[...], preferred_element_type=jnp.float32)
    h = h_ref[...] + _silu(t + b_ref[...])
    ho_ref[...] = h
    hso_ref[...] = h * ns_ref[...]


_layer_call = pl.pallas_call(
    _layer_body,
    out_shape=[
        jax.ShapeDtypeStruct((_NP, _H), jnp.float32),
        jax.ShapeDtypeStruct((_NP, _H), jnp.float32),
    ],
)


def _final_body(aggp_ref, nd_ref, h_ref, W_ref, b_ref,
                Wd1_ref, bd1_ref, Wd2_ref, bd2_ref, out_ref):
    agg = aggp_ref[0] + aggp_ref[1]
    a = agg * nd_ref[...]
    t = jnp.dot(a, W_ref[...], preferred_element_type=jnp.float32)
    h = h_ref[...] + _silu(t + b_ref[...])
    d = _silu(jnp.dot(h, Wd1_ref[...], preferred_element_type=jnp.float32)
              + bd1_ref[...])
    out_ref[...] = (jnp.dot(d, Wd2_ref[...], preferred_element_type=jnp.float32)
                    + bd2_ref[...])


_final_call = pl.pallas_call(
    _final_body,
    out_shape=jax.ShapeDtypeStruct((_NP, _OUT), jnp.float32),
)


def kernel(node_feats, edge_feats, edge_index, Wn, bn, gn, betan, We, be, ge,
           betae, convW, convb, Wd1, bd1, Wd2, bd2):
    src3 = edge_index[0].reshape(_NW, _NCHUNK, _CH)
    dst3 = edge_index[1].reshape(_NW, _NCHUNK, _CH)
    x_p = jnp.pad(node_feats, ((0, _NP - _N), (0, 0)))
    zagg = jnp.zeros((_NP, _H), jnp.float32)
    zdeg = jnp.zeros((_NP, _DEGW), jnp.float32)
    ones_hbm = jnp.ones((_CH, _DEGW), jnp.float32)

    degp = _deg_call(src3, dst3, ones_hbm, zdeg)
    h, hs, ns, nd = _enc_call(x_p, Wn, bn, gn, betan, degp)
    for i in range(_P - 1):
        aggp = _agg_call(hs, src3, dst3, zagg)
        h, hs = _layer_call(aggp, nd, ns, h, convW[i], convb[i])
    aggp = _agg_call(hs, src3, dst3, zagg)
    out_p = _final_call(aggp, nd, h, convW[_P - 1], convb[_P - 1],
                        Wd1, bd1, Wd2, bd2)
    return out_p[:_N]


# speedup vs baseline: 1.2511x; 1.0002x over previous
import functools

import jax
import jax.numpy as jnp
from jax import lax
from jax.experimental import pallas as pl
from jax.experimental.pallas import tpu as pltpu
from jax.experimental.pallas import tpu_sc as plsc

_N = 10000
_E = 320000
_DF = 128
_H = 64
_OUT = 3
_P = 4

_NP = 10240
_NC = 2
_NS = 16
_NW = _NC * _NS
_RPS = _NP // _NS

_EW = _E // _NW
_CH = 80
_NCHUNK = _EW // _CH

_DEGW = 8
_DLAG = 8
_NBUF = 8

_mesh = plsc.VectorSubcoreMesh(core_axis_name="c", subcore_axis_name="s")
_sc_params = pltpu.CompilerParams(use_tc_tiling_on_sc=False)


def _deg_body(src3, dst3, ones_hbm, zdeg, degp, sdeg, idx_v, ones_v, sem):
    cid = lax.axis_index("c")
    sid = lax.axis_index("s")
    zslab = pl.ds(sid * _RPS, _RPS)
    pltpu.sync_copy(zdeg.at[zslab], sdeg.at[zslab])
    pltpu.sync_copy(ones_hbm, ones_v)

    @pl.when(cid == 0)
    def _():
        pltpu.sync_copy(src3.at[2 * sid], idx_v.at[0])
        pltpu.sync_copy(src3.at[2 * sid + 1], idx_v.at[1])

    @pl.when(cid == 1)
    def _():
        pltpu.sync_copy(dst3.at[2 * sid], idx_v.at[0])
        pltpu.sync_copy(dst3.at[2 * sid + 1], idx_v.at[1])

    plsc.subcore_barrier()
    for s in range(2):
        islab = idx_v.at[s]

        def body(i, carry):
            pltpu.async_copy(ones_v, sdeg.at[islab.at[i]], sem, add=True)

            @pl.when(i >= _DLAG)
            def _():
                pltpu.make_async_copy(ones_v, sdeg.at[islab.at[i - _DLAG]],
                                      sem).wait()

            return carry

        lax.fori_loop(0, _NCHUNK, body, 0)
        for i in range(_NCHUNK - _DLAG, _NCHUNK):
            pltpu.make_async_copy(ones_v, sdeg.at[islab.at[i]], sem).wait()
    plsc.subcore_barrier()
    pltpu.sync_copy(sdeg.at[zslab], degp.at[cid, zslab])


_deg_call = pl.kernel(
    _deg_body,
    out_type=jax.ShapeDtypeStruct((_NC, _NP, _DEGW), jnp.float32),
    mesh=_mesh,
    scratch_types=[
        pltpu.VMEM_SHARED((_NP, _DEGW), jnp.float32),
        pltpu.VMEM((2, _NCHUNK, _CH), jnp.int32),
        pltpu.VMEM((_CH, _DEGW), jnp.float32),
        pltpu.SemaphoreType.DMA,
    ],
    compiler_params=_sc_params,
)


def _agg_body(hs, src3, dst3, zagg, aggp, sagg, src_v, dst_v, *rest):
    rows = rest[:_NBUF]
    sems = rest[_NBUF:]
    cid = lax.axis_index("c")
    sid = lax.axis_index("s")
    wid = sid * _NC + cid
    zslab = pl.ds(sid * _RPS, _RPS)
    pltpu.sync_copy(zagg.at[zslab], sagg.at[zslab])
    pltpu.sync_copy(src3.at[wid], src_v)
    pltpu.sync_copy(dst3.at[wid], dst_v)
    plsc.subcore_barrier()

    def gather(i, buf, sem):
        pltpu.async_copy(hs.at[src_v.at[i]], buf, sem)

    def wait(i, buf, sem):
        pltpu.make_async_copy(hs.at[src_v.at[i]], buf, sem).wait()

    def scatter(i, buf):
        pltpu.sync_copy(buf, sagg.at[dst_v.at[i]], add=True)

    bufs = list(zip(rows, sems))
    for b in range(_NBUF):
        gather(b, *bufs[b])

    def body(j, carry):
        for b in range(_NBUF):
            i = _NBUF * j + b
            wait(i, *bufs[b])
            scatter(i, bufs[b][0])

            @pl.when(i + _NBUF < _NCHUNK)
            def _():
                gather(i + _NBUF, *bufs[b])

        return carry

    lax.fori_loop(0, _NCHUNK // _NBUF, body, 0)
    for i in range((_NCHUNK // _NBUF) * _NBUF, _NCHUNK):
        wait(i, *bufs[i % _NBUF])
        scatter(i, bufs[i % _NBUF][0])
    plsc.subcore_barrier()
    pltpu.sync_copy(sagg.at[zslab], aggp.at[cid, zslab])


_agg_call = pl.kernel(
    _agg_body,
    out_type=jax.ShapeDtypeStruct((_NC, _NP, _H), jnp.float32),
    mesh=_mesh,
    scratch_types=[
        pltpu.VMEM_SHARED((_NP, _H), jnp.float32),
        pltpu.VMEM((_NCHUNK, _CH), jnp.int32),
        pltpu.VMEM((_NCHUNK, _CH), jnp.int32),
    ] + [pltpu.VMEM((_CH, _H), jnp.float32)] * _NBUF
      + [pltpu.SemaphoreType.DMA] * _NBUF,
    compiler_params=_sc_params,
)


def _silu(x):
    return x * jax.nn.sigmoid(x)


def _enc_body(x_ref, Wn_ref, bn_ref, gn_ref, betan_ref, degp_ref,
              h_ref, hs_ref, ns_ref, nd_ref):
    t = jnp.dot(x_ref[...], Wn_ref[...], preferred_element_type=jnp.float32)
    t = _silu(t + bn_ref[...])
    mu = jnp.mean(t, axis=-1, keepdims=True)
    var = jnp.mean((t - mu) * (t - mu), axis=-1, keepdims=True)
    h = (t - mu) * lax.rsqrt(var + 1e-5) * gn_ref[...] + betan_ref[...]
    inv = 1.0 / _DEGW
    dego = jnp.sum(degp_ref[0], axis=-1, keepdims=True) * inv
    degi = jnp.sum(degp_ref[1], axis=-1, keepdims=True) * inv
    ns = lax.rsqrt(jnp.maximum(dego, 1.0))
    nd = lax.rsqrt(jnp.maximum(degi, 1.0))
    h_ref[...] = h
    hs_ref[...] = h * ns
    ns_ref[...] = ns
    nd_ref[...] = nd


_enc_call = pl.pallas_call(
    _enc_body,
    out_shape=[
        jax.ShapeDtypeStruct((_NP, _H), jnp.float32),
        jax.ShapeDtypeStruct((_NP, _H), jnp.float32),
        jax.ShapeDtypeStruct((_NP, 1), jnp.float32),
        jax.ShapeDtypeStruct((_NP, 1), jnp.float32),
    ],
)


def _layer_body(aggp_ref, nd_ref, ns_ref, h_ref, W_ref, b_ref,
                ho_ref, hso_ref):
    agg = aggp_ref[0] + aggp_ref[1]
    a = agg * nd_ref[...]
    t = jnp.dot(a, W_ref[...], preferred_element_type=jnp.float32)
    h = h_ref[...] + _silu(t + b_ref[...])
    ho_ref[...] = h
    hso_ref[...] = h * ns_ref[...]


_layer_call = pl.pallas_call(
    _layer_body,
    out_shape=[
        jax.ShapeDtypeStruct((_NP, _H), jnp.float32),
        jax.ShapeDtypeStruct((_NP, _H), jnp.float32),
    ],
)


def _final_body(aggp_ref, nd_ref, h_ref, W_ref, b_ref,
                Wd1_ref, bd1_ref, Wd2_ref, bd2_ref, out_ref):
    agg = aggp_ref[0] + aggp_ref[1]
    a = agg * nd_ref[...]
    t = jnp.dot(a, W_ref[...], preferred_element_type=jnp.float32)
    h = h_ref[...] + _silu(t + b_ref[...])
    d = _silu(jnp.dot(h, Wd1_ref[...], preferred_element_type=jnp.float32)
              + bd1_ref[...])
    out_ref[...] = (jnp.dot(d, Wd2_ref[...], preferred_element_type=jnp.float32)
                    + bd2_ref[...])


_final_call = pl.pallas_call(
    _final_body,
    out_shape=jax.ShapeDtypeStruct((_NP, _OUT), jnp.float32),
)


def kernel(node_feats, edge_feats, edge_index, Wn, bn, gn, betan, We, be, ge,
           betae, convW, convb, Wd1, bd1, Wd2, bd2):
    src3 = edge_index[0].reshape(_NW, _NCHUNK, _CH)
    dst3 = edge_index[1].reshape(_NW, _NCHUNK, _CH)
    x_p = jnp.pad(node_feats, ((0, _NP - _N), (0, 0)))
    zagg = jnp.zeros((_NP, _H), jnp.float32)
    zdeg = jnp.zeros((_NP, _DEGW), jnp.float32)
    ones_hbm = jnp.ones((_CH, _DEGW), jnp.float32)

    degp = _deg_call(src3, dst3, ones_hbm, zdeg)
    h, hs, ns, nd = _enc_call(x_p, Wn, bn, gn, betan, degp)
    for i in range(_P - 1):
        aggp = _agg_call(hs, src3, dst3, zagg)
        h, hs = _layer_call(aggp, nd, ns, h, convW[i], convb[i])
    aggp = _agg_call(hs, src3, dst3, zagg)
    out_p = _final_call(aggp, nd, h, convW[_P - 1], convb[_P - 1],
                        Wd1, bd1, Wd2, bd2)
    return out_p[:_N]
